# baseline (speedup 1.0000x reference)
import functools

import jax
import jax.numpy as jnp
from jax import lax
from jax.experimental import pallas as pl
from jax.experimental.pallas import tpu as pltpu
from jax.experimental.pallas import tpu_sc as plsc

_CHUNK = 32
_RING = 3
_LANES = 16


def _sc_body(total_rows, rows_per_batch, n_workers,
             emb_ref, anchor_ref, out_ref, bufs, anchor_v,
             load_sems, store_sems, anchor_sem):
    rows_per_w = total_rows // n_workers
    n = rows_per_w // _CHUNK
    D = anchor_v.shape[-1]

    wid = lax.axis_index("s") * 2 + lax.axis_index("c")
    base = wid * rows_per_w
    owns_anchor = (base % rows_per_batch) == 0

    acp = pltpu.make_async_copy(anchor_ref, anchor_v, anchor_sem)
    acp.start()
    loads = {}
    stores = {}
    for i in range(min(_RING, n)):
        loads[i] = pltpu.make_async_copy(
            emb_ref.at[pl.ds(base + i * _CHUNK, _CHUNK), :],
            bufs.at[i % _RING],
            load_sems.at[i % _RING],
        )
        loads[i].start()
    acp.wait()
    for i in range(n):
        loads[i].wait()
        if i == 0:
            @pl.when(owns_anchor)
            def _():
                for t in range(D // _LANES):
                    bufs[0, 0, pl.ds(t * _LANES, _LANES)] = (
                        anchor_v[0, pl.ds(t * _LANES, _LANES)]
                    )
        stores[i] = pltpu.make_async_copy(
            bufs.at[i % _RING],
            out_ref.at[pl.ds(base + i * _CHUNK, _CHUNK), :],
            store_sems.at[i % _RING],
        )
        stores[i].start()
        nxt = i + _RING
        if nxt < n:
            stores[i].wait()
            loads[nxt] = pltpu.make_async_copy(
                emb_ref.at[pl.ds(base + nxt * _CHUNK, _CHUNK), :],
                bufs.at[nxt % _RING],
                load_sems.at[nxt % _RING],
            )
            loads[nxt].start()
    for i in range(max(0, n - _RING), n):
        stores[i].wait()


def kernel(token_embeddings, style_anchor):
    B, S, D = token_embeddings.shape
    info = plsc.get_sparse_core_info()
    n_workers = info.num_cores * info.num_subcores
    flat = token_embeddings.reshape(B * S, D)
    mesh = plsc.VectorSubcoreMesh(core_axis_name="c", subcore_axis_name="s")
    fn = pl.kernel(
        functools.partial(_sc_body, B * S, S, n_workers),
        mesh=mesh,
        out_type=jax.ShapeDtypeStruct(flat.shape, flat.dtype),
        scratch_types=[
            pltpu.VMEM((_RING, _CHUNK, D), flat.dtype),
            pltpu.VMEM((1, D), flat.dtype),
            pltpu.SemaphoreType.DMA((_RING,)),
            pltpu.SemaphoreType.DMA((_RING,)),
            pltpu.SemaphoreType.DMA,
        ],
    )
    out = fn(flat, style_anchor)
    return out.reshape(B, S, D)


# speedup vs baseline: 1.0092x; 1.0092x over previous
import functools

import jax
import jax.numpy as jnp
from jax import lax
from jax.experimental import pallas as pl
from jax.experimental.pallas import tpu as pltpu
from jax.experimental.pallas import tpu_sc as plsc

_CHUNK = 16
_RING = 7
_LANES = 16


def _sc_body(total_rows, rows_per_batch, n_workers,
             emb_ref, anchor_ref, out_ref, bufs, anchor_v,
             load_sems, store_sems, anchor_sem):
    rows_per_w = total_rows // n_workers
    n = rows_per_w // _CHUNK
    D = anchor_v.shape[-1]

    wid = lax.axis_index("s") * 2 + lax.axis_index("c")
    base = wid * rows_per_w
    owns_anchor = (base % rows_per_batch) == 0

    acp = pltpu.make_async_copy(anchor_ref, anchor_v, anchor_sem)
    acp.start()
    loads = {}
    stores = {}
    for i in range(min(_RING, n)):
        loads[i] = pltpu.make_async_copy(
            emb_ref.at[pl.ds(base + i * _CHUNK, _CHUNK), :],
            bufs.at[i % _RING],
            load_sems.at[i % _RING],
        )
        loads[i].start()
    acp.wait()
    for i in range(n):
        loads[i].wait()
        if i == 0:
            @pl.when(owns_anchor)
            def _():
                for t in range(D // _LANES):
                    bufs[0, 0, pl.ds(t * _LANES, _LANES)] = (
                        anchor_v[0, pl.ds(t * _LANES, _LANES)]
                    )
        stores[i] = pltpu.make_async_copy(
            bufs.at[i % _RING],
            out_ref.at[pl.ds(base + i * _CHUNK, _CHUNK), :],
            store_sems.at[i % _RING],
        )
        stores[i].start()
        nxt = i + _RING
        if nxt < n:
            stores[i].wait()
            loads[nxt] = pltpu.make_async_copy(
                emb_ref.at[pl.ds(base + nxt * _CHUNK, _CHUNK), :],
                bufs.at[nxt % _RING],
                load_sems.at[nxt % _RING],
            )
            loads[nxt].start()
    for i in range(max(0, n - _RING), n):
        stores[i].wait()


def kernel(token_embeddings, style_anchor):
    B, S, D = token_embeddings.shape
    info = plsc.get_sparse_core_info()
    n_workers = info.num_cores * info.num_subcores
    flat = token_embeddings.reshape(B * S, D)
    mesh = plsc.VectorSubcoreMesh(core_axis_name="c", subcore_axis_name="s")
    fn = pl.kernel(
        functools.partial(_sc_body, B * S, S, n_workers),
        mesh=mesh,
        out_type=jax.ShapeDtypeStruct(flat.shape, flat.dtype),
        scratch_types=[
            pltpu.VMEM((_RING, _CHUNK, D), flat.dtype),
            pltpu.VMEM((1, D), flat.dtype),
            pltpu.SemaphoreType.DMA((_RING,)),
            pltpu.SemaphoreType.DMA((_RING,)),
            pltpu.SemaphoreType.DMA,
        ],
    )
    out = fn(flat, style_anchor)
    return out.reshape(B, S, D)


# speedup vs baseline: 1.6251x; 1.6103x over previous
import jax
import jax.numpy as jnp
from jax.experimental import pallas as pl
from jax.experimental.pallas import tpu as pltpu

_CHUNK = 1024
_RING = 14


def _make_body(total_rows, rows_per_batch):
    n = total_rows // _CHUNK
    anchor_every = rows_per_batch // _CHUNK

    def _body(emb_ref, anchor_ref, out_ref, bufs, anchor_v,
              load_sems, store_sems, anchor_sem):
        acp = pltpu.make_async_copy(anchor_ref, anchor_v, anchor_sem)
        acp.start()
        loads = {}
        stores = {}
        for i in range(min(_RING, n)):
            loads[i] = pltpu.make_async_copy(
                emb_ref.at[pl.ds(i * _CHUNK, _CHUNK), :],
                bufs.at[i % _RING],
                load_sems.at[i % _RING],
            )
            loads[i].start()
        acp.wait()
        for i in range(n):
            loads[i].wait()
            if i % anchor_every == 0:
                bufs[i % _RING, 0:1, :] = anchor_v[...]
            stores[i] = pltpu.make_async_copy(
                bufs.at[i % _RING],
                out_ref.at[pl.ds(i * _CHUNK, _CHUNK), :],
                store_sems.at[i % _RING],
            )
            stores[i].start()
            nxt = i + _RING
            if nxt < n:
                stores[i].wait()
                loads[nxt] = pltpu.make_async_copy(
                    emb_ref.at[pl.ds(nxt * _CHUNK, _CHUNK), :],
                    bufs.at[nxt % _RING],
                    load_sems.at[nxt % _RING],
                )
                loads[nxt].start()
        for i in range(max(0, n - _RING), n):
            stores[i].wait()

    return _body


def kernel(token_embeddings, style_anchor):
    B, S, D = token_embeddings.shape
    flat = token_embeddings.reshape(B * S, D)
    out = pl.pallas_call(
        _make_body(B * S, S),
        in_specs=[
            pl.BlockSpec(memory_space=pl.ANY),
            pl.BlockSpec(memory_space=pl.ANY),
        ],
        out_specs=pl.BlockSpec(memory_space=pl.ANY),
        out_shape=jax.ShapeDtypeStruct(flat.shape, flat.dtype),
        scratch_shapes=[
            pltpu.VMEM((_RING, _CHUNK, D), flat.dtype),
            pltpu.VMEM((1, D), flat.dtype),
            pltpu.SemaphoreType.DMA((_RING,)),
            pltpu.SemaphoreType.DMA((_RING,)),
            pltpu.SemaphoreType.DMA,
        ],
    )(flat, style_anchor)
    return out.reshape(B, S, D)
